# baseline (speedup 1.0000x reference)
import functools

import jax
import jax.numpy as jnp
from jax.experimental import pallas as pl
from jax.experimental.pallas import tpu as pltpu
from jax.experimental.pallas import tpu_sc as plsc
K = 8192
D = 256
N = 8192
BETA = 0.25

TILE_N = 512

_SC_CORES = 2
_SC_SUBCORES = 16
_SC_WORKERS = _SC_CORES * _SC_SUBCORES


def _dist_argmin_kernel(x_ref, ebt_ref, b_ref, idx_ref, lsum_ref, flat_ref):
    xt = x_ref[0]
    flat_ref[...] = xt.T
    a_ref = jnp.sum(xt ** 2, axis=0)[:, None]
    mm = jax.lax.dot_general(
        xt.astype(jnp.bfloat16), ebt_ref[...], (((0,), (0,)), ((), ())),
        preferred_element_type=jnp.float32)
    d = (a_ref[...] + b_ref[...]) - 2.0 * mm
    m = jnp.min(d, axis=1, keepdims=True)
    cols = jax.lax.broadcasted_iota(jnp.int32, d.shape, 1)
    idx = jnp.min(jnp.where(d == m, cols, K), axis=1, keepdims=True)
    idx_ref[...] = idx

    @pl.when(pl.program_id(0) == 0)
    def _():
        lsum_ref[...] = jnp.zeros_like(lsum_ref)

    lsum_ref[...] = lsum_ref[...] + jnp.sum(m).reshape(1, 1)


HW = 1024
NB = 8
_TILES_PER_BATCH = HW // TILE_N


def _tc_dist_argmin(lat3, ebt, b):
    n_loc = NB * HW
    return pl.pallas_call(
        _dist_argmin_kernel,
        grid=(n_loc // TILE_N,),
        in_specs=[
            pl.BlockSpec((1, D, TILE_N),
                         lambda i: (i // _TILES_PER_BATCH, 0,
                                    i % _TILES_PER_BATCH)),
            pl.BlockSpec((D, K), lambda i: (0, 0)),
            pl.BlockSpec((1, K), lambda i: (0, 0)),
        ],
        out_specs=[
            pl.BlockSpec((TILE_N, 1), lambda i: (i, 0)),
            pl.BlockSpec((1, 1), lambda i: (0, 0)),
            pl.BlockSpec((TILE_N, D), lambda i: (i, 0)),
        ],
        out_shape=[
            jax.ShapeDtypeStruct((n_loc, 1), jnp.int32),
            jax.ShapeDtypeStruct((1, 1), jnp.float32),
            jax.ShapeDtypeStruct((n_loc, D), jnp.float32),
        ],
    )(lat3, ebt, b)


def _sc_gather(qdata, idx1d):
    n_loc = idx1d.shape[0]
    rows_per_worker = n_loc // _SC_WORKERS
    mesh = plsc.VectorSubcoreMesh(
        core_axis_name="c", subcore_axis_name="s",
        num_cores=_SC_CORES, num_subcores=_SC_SUBCORES)

    @functools.partial(
        pl.kernel,
        out_type=jax.ShapeDtypeStruct((n_loc, D), jnp.float32),
        mesh=mesh,
        scratch_types=[
            pltpu.VMEM((rows_per_worker,), jnp.int32),
            pltpu.VMEM((rows_per_worker, D), jnp.float32),
            pltpu.SemaphoreType.DMA,
        ],
    )
    def gather_kernel(table_hbm, idx_hbm, out_hbm, idx_v, rows_v, sem):
        wid = jax.lax.axis_index("s") * _SC_CORES + jax.lax.axis_index("c")
        base = wid * rows_per_worker
        pltpu.sync_copy(idx_hbm.at[pl.ds(base, rows_per_worker)], idx_v)
        pltpu.async_copy(table_hbm.at[idx_v], rows_v, sem).wait()
        pltpu.sync_copy(rows_v, out_hbm.at[pl.ds(base, rows_per_worker)])

    return gather_kernel(qdata, idx1d)


def kernel(latents, embedding_weight):
    lat_shape = (NB, 32, 32, D)
    n_loc = NB * HW

    b = jnp.sum(embedding_weight ** 2, axis=1)[None, :]
    ebt = embedding_weight.astype(jnp.bfloat16).T

    lat3 = latents.reshape(NB, D, HW)
    idx, lsum, flat = _tc_dist_argmin(lat3, ebt, b)

    qraw = _sc_gather(embedding_weight, idx.reshape(n_loc))
    q = qraw.astype(jnp.bfloat16).astype(jnp.float32)

    quantized_st = flat + (q - flat)
    out = jnp.transpose(quantized_st.reshape(lat_shape), (0, 3, 1, 2))
    loss = (lsum / jnp.float32(N * D)).reshape(())
    return (out, loss, BETA * loss)


# speedup vs baseline: 1.0949x; 1.0949x over previous
import functools

import jax
import jax.numpy as jnp
from jax.experimental import pallas as pl
from jax.experimental.pallas import tpu as pltpu
from jax.experimental.pallas import tpu_sc as plsc
K = 8192
D = 256
N = 8192
BETA = 0.25

TILE_N = 512

_SC_CORES = 2
_SC_SUBCORES = 16
_SC_WORKERS = _SC_CORES * _SC_SUBCORES


def _dist_argmin_kernel(x_ref, ebt_ref, b_ref, idx_ref, lsum_ref):
    x = x_ref[...]
    a_ref = jnp.sum(x ** 2, axis=1, keepdims=True)
    mm = jax.lax.dot_general(
        x.astype(jnp.bfloat16), ebt_ref[...], (((1,), (0,)), ((), ())),
        preferred_element_type=jnp.float32)
    d = (a_ref[...] + b_ref[...]) - 2.0 * mm
    m = jnp.min(d, axis=1, keepdims=True)
    cols = jax.lax.broadcasted_iota(jnp.int32, d.shape, 1)
    idx = jnp.min(jnp.where(d == m, cols, K), axis=1, keepdims=True)
    idx_ref[...] = idx

    @pl.when(pl.program_id(0) == 0)
    def _():
        lsum_ref[...] = jnp.zeros_like(lsum_ref)

    lsum_ref[...] = lsum_ref[...] + jnp.sum(m).reshape(1, 1)


def _tc_dist_argmin(xb, ebt, b):
    n_loc = xb.shape[0]
    return pl.pallas_call(
        _dist_argmin_kernel,
        grid=(n_loc // TILE_N,),
        in_specs=[
            pl.BlockSpec((TILE_N, D), lambda i: (i, 0)),
            pl.BlockSpec((D, K), lambda i: (0, 0)),
            pl.BlockSpec((1, K), lambda i: (0, 0)),
        ],
        out_specs=[
            pl.BlockSpec((TILE_N, 1), lambda i: (i, 0)),
            pl.BlockSpec((1, 1), lambda i: (0, 0)),
        ],
        out_shape=[
            jax.ShapeDtypeStruct((n_loc, 1), jnp.int32),
            jax.ShapeDtypeStruct((1, 1), jnp.float32),
        ],
    )(xb, ebt, b)


def _sc_gather(qdata, idx1d):
    n_loc = idx1d.shape[0]
    rows_per_worker = n_loc // _SC_WORKERS
    mesh = plsc.VectorSubcoreMesh(
        core_axis_name="c", subcore_axis_name="s",
        num_cores=_SC_CORES, num_subcores=_SC_SUBCORES)

    @functools.partial(
        pl.kernel,
        out_type=jax.ShapeDtypeStruct((n_loc, D), jnp.float32),
        mesh=mesh,
        scratch_types=[
            pltpu.VMEM((rows_per_worker,), jnp.int32),
            pltpu.VMEM((rows_per_worker, D), jnp.float32),
            pltpu.SemaphoreType.DMA,
        ],
    )
    def gather_kernel(table_hbm, idx_hbm, out_hbm, idx_v, rows_v, sem):
        wid = jax.lax.axis_index("s") * _SC_CORES + jax.lax.axis_index("c")
        base = wid * rows_per_worker
        pltpu.sync_copy(idx_hbm.at[pl.ds(base, rows_per_worker)], idx_v)
        pltpu.async_copy(table_hbm.at[idx_v], rows_v, sem).wait()
        pltpu.sync_copy(rows_v, out_hbm.at[pl.ds(base, rows_per_worker)])

    return gather_kernel(qdata, idx1d)


def kernel(latents, embedding_weight):
    x = jnp.transpose(latents, (0, 2, 3, 1))
    lat_shape = x.shape
    flat = x.reshape(-1, D)
    n_loc = flat.shape[0]

    b = jnp.sum(embedding_weight ** 2, axis=1)[None, :]
    ebt = embedding_weight.astype(jnp.bfloat16).T

    idx, lsum = _tc_dist_argmin(flat, ebt, b)

    qraw = _sc_gather(embedding_weight, idx.reshape(n_loc))
    q = qraw.astype(jnp.bfloat16).astype(jnp.float32)

    quantized_st = flat + jax.lax.stop_gradient(q - flat)
    out = jnp.transpose(quantized_st.reshape(lat_shape), (0, 3, 1, 2))
    loss = (lsum / jnp.float32(N * D)).reshape(())
    return (out, loss, BETA * loss)


# speedup vs baseline: 1.0956x; 1.0006x over previous
import functools

import jax
import jax.numpy as jnp
from jax.experimental import pallas as pl
from jax.experimental.pallas import tpu as pltpu
from jax.experimental.pallas import tpu_sc as plsc
K = 8192
D = 256
N = 8192
BETA = 0.25

TILE_N = 512

_SC_CORES = 2
_SC_SUBCORES = 16
_SC_WORKERS = _SC_CORES * _SC_SUBCORES


def _dist_argmin_kernel(x_ref, ebt_ref, b_ref, idx_ref, lsum_ref):
    x = x_ref[...]
    a = jnp.sum(x ** 2, axis=1, keepdims=True)
    mm = jax.lax.dot_general(
        x.astype(jnp.bfloat16), ebt_ref[...], (((1,), (0,)), ((), ())),
        preferred_element_type=jnp.float32)
    d = (a + b_ref[...]) - 2.0 * mm
    m = jnp.min(d, axis=1, keepdims=True)
    cols = jax.lax.broadcasted_iota(jnp.int32, d.shape, 1)
    idx = jnp.min(jnp.where(d == m, cols, K), axis=1, keepdims=True)
    idx_ref[...] = idx

    @pl.when(pl.program_id(0) == 0)
    def _():
        lsum_ref[...] = jnp.zeros_like(lsum_ref)

    lsum_ref[...] = lsum_ref[...] + jnp.sum(m).reshape(1, 1)


def _tc_dist_argmin(xb, ebt, b):
    n_loc = xb.shape[0]
    return pl.pallas_call(
        _dist_argmin_kernel,
        grid=(n_loc // TILE_N,),
        in_specs=[
            pl.BlockSpec((TILE_N, D), lambda i: (i, 0)),
            pl.BlockSpec((D, K), lambda i: (0, 0)),
            pl.BlockSpec((1, K), lambda i: (0, 0)),
        ],
        out_specs=[
            pl.BlockSpec((TILE_N, 1), lambda i: (i, 0)),
            pl.BlockSpec((1, 1), lambda i: (0, 0)),
        ],
        out_shape=[
            jax.ShapeDtypeStruct((n_loc, 1), jnp.int32),
            jax.ShapeDtypeStruct((1, 1), jnp.float32),
        ],
    )(xb, ebt, b)


def _sc_gather(qdata, idx1d):
    n_loc = idx1d.shape[0]
    rows_per_worker = n_loc // _SC_WORKERS
    mesh = plsc.VectorSubcoreMesh(
        core_axis_name="c", subcore_axis_name="s",
        num_cores=_SC_CORES, num_subcores=_SC_SUBCORES)

    @functools.partial(
        pl.kernel,
        out_type=jax.ShapeDtypeStruct((n_loc, D), jnp.float32),
        mesh=mesh,
        scratch_types=[
            pltpu.VMEM((rows_per_worker,), jnp.int32),
            pltpu.VMEM((rows_per_worker, D), jnp.float32),
            pltpu.SemaphoreType.DMA,
        ],
    )
    def gather_kernel(table_hbm, idx_hbm, out_hbm, idx_v, rows_v, sem):
        wid = jax.lax.axis_index("s") * _SC_CORES + jax.lax.axis_index("c")
        base = wid * rows_per_worker
        pltpu.sync_copy(idx_hbm.at[pl.ds(base, rows_per_worker)], idx_v)
        pltpu.async_copy(table_hbm.at[idx_v], rows_v, sem).wait()
        pltpu.sync_copy(rows_v, out_hbm.at[pl.ds(base, rows_per_worker)])

    return gather_kernel(qdata, idx1d)


def kernel(latents, embedding_weight):
    x = jnp.transpose(latents, (0, 2, 3, 1))
    lat_shape = x.shape
    flat = x.reshape(-1, D)
    n_loc = flat.shape[0]

    b = jnp.sum(embedding_weight ** 2, axis=1)[None, :]
    ebt = embedding_weight.astype(jnp.bfloat16).T

    idx, lsum = _tc_dist_argmin(flat, ebt, b)

    qraw = _sc_gather(embedding_weight, idx.reshape(n_loc))
    q = qraw.astype(jnp.bfloat16).astype(jnp.float32)

    quantized_st = flat + jax.lax.stop_gradient(q - flat)
    out = jnp.transpose(quantized_st.reshape(lat_shape), (0, 3, 1, 2))
    loss = (lsum / jnp.float32(N * D)).reshape(())
    return (out, loss, BETA * loss)
